# baseline (speedup 1.0000x reference)
import functools

import jax
import jax.numpy as jnp
from jax.experimental import pallas as pl
from jax.experimental.pallas import tpu as pltpu

TOP_K = 2
BLK = 256


def _ffn_block_kernel(eidx_ref, rowtok_ref, x_ref, W1_ref, b1_ref, W2_ref,
                      b2_ref, y_ref, xg_ref, h_ref):
    b = pl.program_id(0)
    B = y_ref.shape[0]

    def gather_row(r, _):
        t = rowtok_ref[b * B + r]
        xg_ref[pl.ds(r, 1)] = x_ref[pl.ds(t, 1)]
        return 0

    jax.lax.fori_loop(0, B, gather_row, 0)
    xv = xg_ref[...].reshape(B, x_ref.shape[2])
    h = jnp.dot(xv, W1_ref[0], preferred_element_type=jnp.float32)
    h_ref[...] = jax.nn.gelu(h + b1_ref[0]).astype(jnp.bfloat16)
    y = jnp.dot(h_ref[...], W2_ref[0], preferred_element_type=jnp.float32)
    y_ref[...] = y + b2_ref[0]


def _grouped_ffn(x, W1_all, b1_all, W2_all, b2_all, eidx, rowtok, nb, blk):
    T, D = x.shape
    F = W1_all.shape[2]
    grid_spec = pltpu.PrefetchScalarGridSpec(
        num_scalar_prefetch=2,
        grid=(nb,),
        in_specs=[
            pl.BlockSpec((T, 1, D), lambda b, e, rt: (0, 0, 0)),
            pl.BlockSpec((1, D, F), lambda b, e, rt: (e[b], 0, 0)),
            pl.BlockSpec((1, 1, F), lambda b, e, rt: (e[b], 0, 0)),
            pl.BlockSpec((1, F, D), lambda b, e, rt: (e[b], 0, 0)),
            pl.BlockSpec((1, 1, D), lambda b, e, rt: (e[b], 0, 0)),
        ],
        out_specs=pl.BlockSpec((blk, D), lambda b, e, rt: (b, 0)),
        scratch_shapes=[pltpu.VMEM((blk, 1, D), jnp.bfloat16),
                        pltpu.VMEM((blk, F), jnp.bfloat16)],
    )
    return pl.pallas_call(
        _ffn_block_kernel,
        grid_spec=grid_spec,
        out_shape=jax.ShapeDtypeStruct((nb * blk, D), jnp.float32),
    )(eidx, rowtok, x[:, None, :], W1_all, b1_all[:, None, :], W2_all,
      b2_all[:, None, :])


def kernel(x, shared_W1, shared_b1, shared_W2, shared_b2, routed_W1,
           routed_b1, routed_W2, routed_b2, gate_W, gate_b):
    T, D = x.shape
    E = routed_W1.shape[0]
    S = shared_W1.shape[0]
    nb_r = (T * TOP_K) // BLK + E - 1
    nb_s = (T * S) // BLK
    nb = nb_r + nb_s
    np_r = nb_r * BLK

    logits = x @ gate_W + gate_b
    probs = jax.nn.softmax(logits, axis=-1)
    topk_probs, topk_idx = jax.lax.top_k(probs, TOP_K)
    ef = topk_idx.reshape(-1).astype(jnp.int32)
    oh = jax.nn.one_hot(ef, E, dtype=jnp.int32)
    counts = jnp.sum(oh, axis=0)
    rank = (jnp.cumsum(oh, axis=0) - oh)
    rank = jnp.sum(rank * oh, axis=1)
    blocks_per_e = (counts + BLK - 1) // BLK
    padded_off = jnp.cumsum(blocks_per_e) * BLK - blocks_per_e * BLK
    pos = padded_off[ef] + rank
    eidx = jnp.searchsorted(jnp.cumsum(blocks_per_e),
                            jnp.arange(nb_r), side='right').astype(jnp.int32)
    eidx = jnp.minimum(eidx, E - 1)
    eidx = jnp.concatenate([
        eidx,
        jnp.repeat(jnp.arange(E, E + S, dtype=jnp.int32), T // BLK),
    ])
    tok = jnp.arange(T * TOP_K, dtype=jnp.int32) // TOP_K
    rowtok_r = jnp.zeros((np_r,), jnp.int32).at[pos].set(tok)
    rowtok = jnp.concatenate(
        [rowtok_r] + [jnp.arange(T, dtype=jnp.int32)] * S)

    W1_all = jnp.concatenate([routed_W1, shared_W1]).astype(jnp.bfloat16)
    b1_all = jnp.concatenate([routed_b1, shared_b1])
    W2_all = jnp.concatenate([routed_W2, shared_W2]).astype(jnp.bfloat16)
    b2_all = jnp.concatenate([routed_b2, shared_b2])
    y = _grouped_ffn(x.astype(jnp.bfloat16), W1_all, b1_all, W2_all, b2_all,
                     eidx, rowtok, nb, BLK)

    pos_tk = pos.reshape(T, TOP_K)
    w = topk_probs
    out = (w[:, 0:1] * y[pos_tk[:, 0]] + w[:, 1:2] * y[pos_tk[:, 1]]
           + y[np_r:np_r + T] + y[np_r + T:np_r + 2 * T])
    return out


# speedup vs baseline: 1.6964x; 1.6964x over previous
import functools

import jax
import jax.numpy as jnp
from jax.experimental import pallas as pl
from jax.experimental.pallas import tpu as pltpu

TOP_K = 2
BLK = 256
SBLK = 256


def _routed_kernel(eidx_ref, rowtok_ref, x_ref, W1_ref, b1_ref, W2_ref,
                   b2_ref, y_ref, w1c_ref, w2c_ref, xg_ref, h_ref,
                   sem1, sem2):
    b = pl.program_id(0)
    B = y_ref.shape[0]
    e = eidx_ref[b]
    prev = eidx_ref[jnp.maximum(b - 1, 0)]
    need_load = jnp.logical_or(b == 0, e != prev)

    @pl.when(need_load)
    def _():
        pltpu.make_async_copy(W1_ref.at[e], w1c_ref, sem1).start()
        pltpu.make_async_copy(W2_ref.at[e], w2c_ref, sem2).start()

    def gather_row(r, _):
        t = rowtok_ref[b * B + r]
        xg_ref[pl.ds(r, 1)] = x_ref[pl.ds(t, 1)]
        return 0

    jax.lax.fori_loop(0, B, gather_row, 0)

    @pl.when(need_load)
    def _():
        pltpu.make_async_copy(W1_ref.at[e], w1c_ref, sem1).wait()

    xv = xg_ref[...].reshape(B, x_ref.shape[2])
    h = jnp.dot(xv, w1c_ref[...], preferred_element_type=jnp.float32)
    h_ref[...] = jax.nn.gelu(h + b1_ref[0])

    @pl.when(need_load)
    def _():
        pltpu.make_async_copy(W2_ref.at[e], w2c_ref, sem2).wait()

    y = jnp.dot(h_ref[...], w2c_ref[...], preferred_element_type=jnp.float32)
    y_ref[...] = y + b2_ref[0]


def _shared_kernel(x_ref, W1_ref, b1_ref, W2_ref, b2_ref, o_ref,
                   w1c_ref, w2c_ref, h_ref, sem1, sem2):
    e = pl.program_id(0)
    b = pl.program_id(1)
    B = x_ref.shape[0]
    need_load = b == 0

    @pl.when(need_load)
    def _():
        pltpu.make_async_copy(W1_ref.at[e], w1c_ref, sem1).start()
        pltpu.make_async_copy(W2_ref.at[e], w2c_ref, sem2).start()

    @pl.when(need_load)
    def _():
        pltpu.make_async_copy(W1_ref.at[e], w1c_ref, sem1).wait()

    h = jnp.dot(x_ref[...], w1c_ref[...], preferred_element_type=jnp.float32)
    h_ref[...] = jax.nn.gelu(h + b1_ref[0])

    @pl.when(need_load)
    def _():
        pltpu.make_async_copy(W2_ref.at[e], w2c_ref, sem2).wait()

    y = jnp.dot(h_ref[...], w2c_ref[...], preferred_element_type=jnp.float32)
    y = y + b2_ref[0]
    row = pl.multiple_of(b * B, B)

    @pl.when(e == 0)
    def _():
        o_ref[pl.ds(row, B), :] = y

    @pl.when(e > 0)
    def _():
        o_ref[pl.ds(row, B), :] = o_ref[pl.ds(row, B), :] + y


def _routed_ffn(x, W1, b1, W2, b2, eidx, rowtok, nb, blk):
    T, _, D = x.shape
    F = W1.shape[2]
    grid_spec = pltpu.PrefetchScalarGridSpec(
        num_scalar_prefetch=2,
        grid=(nb,),
        in_specs=[
            pl.BlockSpec((T, 1, D), lambda b, e, rt: (0, 0, 0)),
            pl.BlockSpec(memory_space=pl.ANY),
            pl.BlockSpec((1, 1, F), lambda b, e, rt: (e[b], 0, 0)),
            pl.BlockSpec(memory_space=pl.ANY),
            pl.BlockSpec((1, 1, D), lambda b, e, rt: (e[b], 0, 0)),
        ],
        out_specs=pl.BlockSpec((blk, D), lambda b, e, rt: (b, 0)),
        scratch_shapes=[pltpu.VMEM((D, F), jnp.float32),
                        pltpu.VMEM((F, D), jnp.float32),
                        pltpu.VMEM((blk, 1, D), jnp.float32),
                        pltpu.VMEM((blk, F), jnp.float32),
                        pltpu.SemaphoreType.DMA,
                        pltpu.SemaphoreType.DMA],
    )
    return pl.pallas_call(
        _routed_kernel,
        grid_spec=grid_spec,
        out_shape=jax.ShapeDtypeStruct((nb * blk, D), jnp.float32),
        compiler_params=pltpu.CompilerParams(
            vmem_limit_bytes=60 * 1024 * 1024),
    )(eidx, rowtok, x, W1, b1[:, None, :], W2, b2[:, None, :])


def _shared_ffn(x, W1, b1, W2, b2):
    T, D = x.shape
    S, _, F = W1.shape
    nb = T // SBLK
    return pl.pallas_call(
        _shared_kernel,
        grid=(S, nb),
        in_specs=[
            pl.BlockSpec((SBLK, D), lambda e, b: (b, 0)),
            pl.BlockSpec(memory_space=pl.ANY),
            pl.BlockSpec((1, 1, F), lambda e, b: (e, 0, 0)),
            pl.BlockSpec(memory_space=pl.ANY),
            pl.BlockSpec((1, 1, D), lambda e, b: (e, 0, 0)),
        ],
        out_specs=pl.BlockSpec((T, D), lambda e, b: (0, 0)),
        out_shape=jax.ShapeDtypeStruct((T, D), jnp.float32),
        scratch_shapes=[pltpu.VMEM((D, F), jnp.float32),
                        pltpu.VMEM((F, D), jnp.float32),
                        pltpu.VMEM((SBLK, F), jnp.float32),
                        pltpu.SemaphoreType.DMA,
                        pltpu.SemaphoreType.DMA],
        compiler_params=pltpu.CompilerParams(
            vmem_limit_bytes=60 * 1024 * 1024),
    )(x, W1, b1[:, None, :], W2, b2[:, None, :])


def kernel(x, shared_W1, shared_b1, shared_W2, shared_b2, routed_W1,
           routed_b1, routed_W2, routed_b2, gate_W, gate_b):
    T, D = x.shape
    E = routed_W1.shape[0]
    nb_r = (T * TOP_K) // BLK + E - 1
    np_r = nb_r * BLK

    logits = x @ gate_W + gate_b
    probs = jax.nn.softmax(logits, axis=-1)
    topk_probs, topk_idx = jax.lax.top_k(probs, TOP_K)
    ef = topk_idx.reshape(-1).astype(jnp.int32)
    oh = jax.nn.one_hot(ef, E, dtype=jnp.int32)
    counts = jnp.sum(oh, axis=0)
    rank = (jnp.cumsum(oh, axis=0) - oh)
    rank = jnp.sum(rank * oh, axis=1)
    blocks_per_e = (counts + BLK - 1) // BLK
    padded_off = jnp.cumsum(blocks_per_e) * BLK - blocks_per_e * BLK
    pos = padded_off[ef] + rank
    eidx = jnp.searchsorted(jnp.cumsum(blocks_per_e),
                            jnp.arange(nb_r), side='right').astype(jnp.int32)
    eidx = jnp.minimum(eidx, E - 1)
    tok = jnp.arange(T * TOP_K, dtype=jnp.int32) // TOP_K
    rowtok = jnp.zeros((np_r,), jnp.int32).at[pos].set(tok)

    y_r = _routed_ffn(x[:, None, :], routed_W1, routed_b1, routed_W2,
                      routed_b2, eidx, rowtok, nb_r, BLK)
    y_s = _shared_ffn(x, shared_W1, shared_b1, shared_W2, shared_b2)

    pos_tk = pos.reshape(T, TOP_K)
    w = topk_probs
    out = (w[:, 0:1] * y_r[pos_tk[:, 0]] + w[:, 1:2] * y_r[pos_tk[:, 1]]
           + y_s)
    return out


# speedup vs baseline: 1.7091x; 1.0075x over previous
import functools

import jax
import jax.numpy as jnp
from jax import lax
from jax.experimental import pallas as pl
from jax.experimental.pallas import tpu as pltpu
from jax.experimental.pallas import tpu_sc as plsc

TOP_K = 2
BLK = 256
SBLK = 256
NC = 2
NS = 16
NW = NC * NS
LANES = 16


def _routed_kernel(eidx_ref, xp_ref, W1_ref, b1_ref, W2_ref,
                   b2_ref, y_ref, w1c_ref, w2c_ref, h_ref, sem1, sem2):
    b = pl.program_id(0)
    e = eidx_ref[b]
    prev = eidx_ref[jnp.maximum(b - 1, 0)]
    need_load = jnp.logical_or(b == 0, e != prev)

    @pl.when(need_load)
    def _():
        pltpu.make_async_copy(W1_ref.at[e], w1c_ref, sem1).start()
        pltpu.make_async_copy(W2_ref.at[e], w2c_ref, sem2).start()

    @pl.when(need_load)
    def _():
        pltpu.make_async_copy(W1_ref.at[e], w1c_ref, sem1).wait()

    h = jnp.dot(xp_ref[...], w1c_ref[...], preferred_element_type=jnp.float32)
    h_ref[...] = jax.nn.gelu(h + b1_ref[0])

    @pl.when(need_load)
    def _():
        pltpu.make_async_copy(W2_ref.at[e], w2c_ref, sem2).wait()

    y = jnp.dot(h_ref[...], w2c_ref[...], preferred_element_type=jnp.float32)
    y_ref[...] = y + b2_ref[0]


def _shared_kernel(x_ref, W1_ref, b1_ref, W2_ref, b2_ref, o_ref,
                   w1c_ref, w2c_ref, h_ref, sem1, sem2):
    e = pl.program_id(0)
    b = pl.program_id(1)
    B = x_ref.shape[0]
    need_load = b == 0

    @pl.when(need_load)
    def _():
        pltpu.make_async_copy(W1_ref.at[e], w1c_ref, sem1).start()
        pltpu.make_async_copy(W2_ref.at[e], w2c_ref, sem2).start()

    @pl.when(need_load)
    def _():
        pltpu.make_async_copy(W1_ref.at[e], w1c_ref, sem1).wait()

    h = jnp.dot(x_ref[...], w1c_ref[...], preferred_element_type=jnp.float32)
    h_ref[...] = jax.nn.gelu(h + b1_ref[0])

    @pl.when(need_load)
    def _():
        pltpu.make_async_copy(W2_ref.at[e], w2c_ref, sem2).wait()

    y = jnp.dot(h_ref[...], w2c_ref[...], preferred_element_type=jnp.float32)
    y = y + b2_ref[0]
    row = pl.multiple_of(b * B, B)

    @pl.when(e == 0)
    def _():
        o_ref[pl.ds(row, B), :] = y

    @pl.when(e > 0)
    def _():
        o_ref[pl.ds(row, B), :] = o_ref[pl.ds(row, B), :] + y


def _routed_ffn(xp, W1, b1, W2, b2, eidx, nb, blk):
    D = xp.shape[1]
    F = W1.shape[2]
    grid_spec = pltpu.PrefetchScalarGridSpec(
        num_scalar_prefetch=1,
        grid=(nb,),
        in_specs=[
            pl.BlockSpec((blk, D), lambda b, e: (b, 0)),
            pl.BlockSpec(memory_space=pl.ANY),
            pl.BlockSpec((1, 1, F), lambda b, e: (e[b], 0, 0)),
            pl.BlockSpec(memory_space=pl.ANY),
            pl.BlockSpec((1, 1, D), lambda b, e: (e[b], 0, 0)),
        ],
        out_specs=pl.BlockSpec((blk, D), lambda b, e: (b, 0)),
        scratch_shapes=[pltpu.VMEM((D, F), jnp.float32),
                        pltpu.VMEM((F, D), jnp.float32),
                        pltpu.VMEM((blk, F), jnp.float32),
                        pltpu.SemaphoreType.DMA,
                        pltpu.SemaphoreType.DMA],
    )
    return pl.pallas_call(
        _routed_kernel,
        grid_spec=grid_spec,
        out_shape=jax.ShapeDtypeStruct((nb * blk, D), jnp.float32),
        compiler_params=pltpu.CompilerParams(
            vmem_limit_bytes=60 * 1024 * 1024),
    )(eidx, xp, W1, b1[:, None, :], W2, b2[:, None, :])


def _shared_ffn(x, W1, b1, W2, b2):
    T, D = x.shape
    S, _, F = W1.shape
    nb = T // SBLK
    return pl.pallas_call(
        _shared_kernel,
        grid=(S, nb),
        in_specs=[
            pl.BlockSpec((SBLK, D), lambda e, b: (b, 0)),
            pl.BlockSpec(memory_space=pl.ANY),
            pl.BlockSpec((1, 1, F), lambda e, b: (e, 0, 0)),
            pl.BlockSpec(memory_space=pl.ANY),
            pl.BlockSpec((1, 1, D), lambda e, b: (e, 0, 0)),
        ],
        out_specs=pl.BlockSpec((T, D), lambda e, b: (0, 0)),
        out_shape=jax.ShapeDtypeStruct((T, D), jnp.float32),
        scratch_shapes=[pltpu.VMEM((D, F), jnp.float32),
                        pltpu.VMEM((F, D), jnp.float32),
                        pltpu.VMEM((SBLK, F), jnp.float32),
                        pltpu.SemaphoreType.DMA,
                        pltpu.SemaphoreType.DMA],
        compiler_params=pltpu.CompilerParams(
            vmem_limit_bytes=60 * 1024 * 1024),
    )(x, W1, b1[:, None, :], W2, b2[:, None, :])


def _sc_dispatch(x, pos0, pos1, np_r):
    T, D = x.shape
    tpw = T // NW
    mesh = plsc.VectorSubcoreMesh(core_axis_name="c", subcore_axis_name="s")

    @functools.partial(
        pl.kernel, mesh=mesh,
        out_type=jax.ShapeDtypeStruct((np_r, D), jnp.float32),
        scratch_types=[pltpu.VMEM((tpw,), jnp.int32),
                       pltpu.VMEM((tpw,), jnp.int32),
                       pltpu.VMEM((tpw, D), jnp.float32),
                       pltpu.SemaphoreType.DMA],
    )
    def k(x_hbm, p0_hbm, p1_hbm, xp_hbm, i0_v, i1_v, rows_v, sem):
        wid = lax.axis_index("s") * NC + lax.axis_index("c")
        base = wid * tpw
        pltpu.sync_copy(p0_hbm.at[pl.ds(base, tpw)], i0_v)
        pltpu.sync_copy(p1_hbm.at[pl.ds(base, tpw)], i1_v)
        pltpu.sync_copy(x_hbm.at[pl.ds(base, tpw)], rows_v)
        pltpu.async_copy(rows_v, xp_hbm.at[i0_v], sem).wait()
        pltpu.async_copy(rows_v, xp_hbm.at[i1_v], sem).wait()

    return k(x, pos0, pos1)


def _sc_combine(y_r, y_s, pos0, pos1, w0, w1):
    T, D = y_s.shape
    tpw = T // NW
    ch = 16
    mesh = plsc.VectorSubcoreMesh(core_axis_name="c", subcore_axis_name="s")

    @functools.partial(
        pl.kernel, mesh=mesh,
        out_type=jax.ShapeDtypeStruct((T, D), jnp.float32),
        scratch_types=[pltpu.VMEM((tpw,), jnp.int32),
                       pltpu.VMEM((tpw,), jnp.int32),
                       pltpu.VMEM((tpw, LANES), jnp.float32),
                       pltpu.VMEM((tpw, LANES), jnp.float32),
                       pltpu.VMEM((ch, D), jnp.float32),
                       pltpu.VMEM((ch, D), jnp.float32),
                       pltpu.VMEM((ch, D), jnp.float32),
                       pltpu.SemaphoreType.DMA],
    )
    def k(yr_hbm, ys_hbm, p0_hbm, p1_hbm, w0_hbm, w1_hbm, out_hbm,
          i0_v, i1_v, w0_v, w1_v, r0_v, r1_v, ys_v, sem):
        wid = lax.axis_index("s") * NC + lax.axis_index("c")
        base = wid * tpw
        pltpu.sync_copy(p0_hbm.at[pl.ds(base, tpw)], i0_v)
        pltpu.sync_copy(p1_hbm.at[pl.ds(base, tpw)], i1_v)
        pltpu.sync_copy(w0_hbm.at[pl.ds(base, tpw)], w0_v)
        pltpu.sync_copy(w1_hbm.at[pl.ds(base, tpw)], w1_v)
        for c in range(tpw // ch):
            off = c * ch
            pltpu.async_copy(yr_hbm.at[i0_v.at[pl.ds(off, ch)]], r0_v,
                             sem).wait()
            pltpu.async_copy(yr_hbm.at[i1_v.at[pl.ds(off, ch)]], r1_v,
                             sem).wait()
            pltpu.sync_copy(ys_hbm.at[pl.ds(base + off, ch)], ys_v)
            for r in range(ch):
                a0 = w0_v[off + r]
                a1 = w1_v[off + r]

                def body(j, _):
                    jj = j * LANES
                    r0_v[r, pl.ds(jj, LANES)] = (
                        a0 * r0_v[r, pl.ds(jj, LANES)]
                        + a1 * r1_v[r, pl.ds(jj, LANES)]
                        + ys_v[r, pl.ds(jj, LANES)])
                    return 0

                lax.fori_loop(0, D // LANES, body, 0)
            pltpu.sync_copy(r0_v, out_hbm.at[pl.ds(base + off, ch)])

    return k(y_r, y_s, pos0, pos1, w0, w1)


def kernel(x, shared_W1, shared_b1, shared_W2, shared_b2, routed_W1,
           routed_b1, routed_W2, routed_b2, gate_W, gate_b):
    T, D = x.shape
    E = routed_W1.shape[0]
    nb_r = (T * TOP_K) // BLK + E - 1
    np_r = nb_r * BLK

    logits = x @ gate_W + gate_b
    probs = jax.nn.softmax(logits, axis=-1)
    topk_probs, topk_idx = jax.lax.top_k(probs, TOP_K)
    ef = topk_idx.reshape(-1).astype(jnp.int32)
    oh = jax.nn.one_hot(ef, E, dtype=jnp.int32)
    counts = jnp.sum(oh, axis=0)
    rank = (jnp.cumsum(oh, axis=0) - oh)
    rank = jnp.sum(rank * oh, axis=1)
    blocks_per_e = (counts + BLK - 1) // BLK
    padded_off = jnp.cumsum(blocks_per_e) * BLK - blocks_per_e * BLK
    pos = padded_off[ef] + rank
    eidx = jnp.searchsorted(jnp.cumsum(blocks_per_e),
                            jnp.arange(nb_r), side='right').astype(jnp.int32)
    eidx = jnp.minimum(eidx, E - 1)
    pos_tk = pos.reshape(T, TOP_K)
    pos0 = pos_tk[:, 0]
    pos1 = pos_tk[:, 1]
    w0 = jnp.broadcast_to(topk_probs[:, 0:1], (T, LANES))
    w1 = jnp.broadcast_to(topk_probs[:, 1:2], (T, LANES))

    xp = _sc_dispatch(x, pos0, pos1, np_r)

    y_r = _routed_ffn(xp, routed_W1, routed_b1, routed_W2, routed_b2,
                      eidx, nb_r, BLK)
    y_s = _shared_ffn(x, shared_W1, shared_b1, shared_W2, shared_b2)

    return _sc_combine(y_r, y_s, pos0, pos1, w0, w1)


# speedup vs baseline: 1.8133x; 1.0610x over previous
import functools

import jax
import jax.numpy as jnp
from jax import lax
from jax.experimental import pallas as pl
from jax.experimental.pallas import tpu as pltpu
from jax.experimental.pallas import tpu_sc as plsc

TOP_K = 2
BLK = 256
SBLK = 256
NC = 2
NS = 16
NW = NC * NS
LANES = 16


def _routed_kernel(eidx_ref, xp_ref, W1_ref, b1_ref, W2_ref,
                   b2_ref, y_ref, w1c_ref, w2c_ref, h_ref, sem1, sem2):
    b = pl.program_id(0)
    e = eidx_ref[b]
    prev = eidx_ref[jnp.maximum(b - 1, 0)]
    need_load = jnp.logical_or(b == 0, e != prev)

    @pl.when(need_load)
    def _():
        pltpu.make_async_copy(W1_ref.at[e], w1c_ref, sem1).start()
        pltpu.make_async_copy(W2_ref.at[e], w2c_ref, sem2).start()

    @pl.when(need_load)
    def _():
        pltpu.make_async_copy(W1_ref.at[e], w1c_ref, sem1).wait()

    h = jnp.dot(xp_ref[...], w1c_ref[...], preferred_element_type=jnp.float32)
    h_ref[...] = jax.nn.gelu(h + b1_ref[0])

    @pl.when(need_load)
    def _():
        pltpu.make_async_copy(W2_ref.at[e], w2c_ref, sem2).wait()

    y = jnp.dot(h_ref[...], w2c_ref[...], preferred_element_type=jnp.float32)
    y_ref[...] = y + b2_ref[0]


def _shared_kernel(x_ref, W1_ref, b1_ref, W2_ref, b2_ref, o_ref,
                   w1c_ref, w2c_ref, h_ref, sem1, sem2):
    e = pl.program_id(0)
    b = pl.program_id(1)
    B = x_ref.shape[0]
    need_load = b == 0

    @pl.when(need_load)
    def _():
        pltpu.make_async_copy(W1_ref.at[e], w1c_ref, sem1).start()
        pltpu.make_async_copy(W2_ref.at[e], w2c_ref, sem2).start()

    @pl.when(need_load)
    def _():
        pltpu.make_async_copy(W1_ref.at[e], w1c_ref, sem1).wait()

    h = jnp.dot(x_ref[...], w1c_ref[...], preferred_element_type=jnp.float32)
    h_ref[...] = jax.nn.gelu(h + b1_ref[0])

    @pl.when(need_load)
    def _():
        pltpu.make_async_copy(W2_ref.at[e], w2c_ref, sem2).wait()

    y = jnp.dot(h_ref[...], w2c_ref[...], preferred_element_type=jnp.float32)
    y = y + b2_ref[0]
    row = pl.multiple_of(b * B, B)

    @pl.when(e == 0)
    def _():
        o_ref[pl.ds(row, B), :] = y

    @pl.when(e > 0)
    def _():
        o_ref[pl.ds(row, B), :] = o_ref[pl.ds(row, B), :] + y


def _routed_ffn(xp, W1, b1, W2, b2, eidx, nb, blk):
    D = xp.shape[1]
    F = W1.shape[2]
    grid_spec = pltpu.PrefetchScalarGridSpec(
        num_scalar_prefetch=1,
        grid=(nb,),
        in_specs=[
            pl.BlockSpec((blk, D), lambda b, e: (b, 0)),
            pl.BlockSpec(memory_space=pl.ANY),
            pl.BlockSpec((1, 1, F), lambda b, e: (e[b], 0, 0)),
            pl.BlockSpec(memory_space=pl.ANY),
            pl.BlockSpec((1, 1, D), lambda b, e: (e[b], 0, 0)),
        ],
        out_specs=pl.BlockSpec((blk, D), lambda b, e: (b, 0)),
        scratch_shapes=[pltpu.VMEM((D, F), jnp.float32),
                        pltpu.VMEM((F, D), jnp.float32),
                        pltpu.VMEM((blk, F), jnp.float32),
                        pltpu.SemaphoreType.DMA,
                        pltpu.SemaphoreType.DMA],
    )
    return pl.pallas_call(
        _routed_kernel,
        grid_spec=grid_spec,
        out_shape=jax.ShapeDtypeStruct((nb * blk, D), jnp.float32),
        compiler_params=pltpu.CompilerParams(
            vmem_limit_bytes=60 * 1024 * 1024),
    )(eidx, xp, W1, b1[:, None, :], W2, b2[:, None, :])


def _shared_ffn(x, W1, b1, W2, b2):
    T, D = x.shape
    S, _, F = W1.shape
    nb = T // SBLK
    return pl.pallas_call(
        _shared_kernel,
        grid=(S, nb),
        in_specs=[
            pl.BlockSpec((SBLK, D), lambda e, b: (b, 0)),
            pl.BlockSpec(memory_space=pl.ANY),
            pl.BlockSpec((1, 1, F), lambda e, b: (e, 0, 0)),
            pl.BlockSpec(memory_space=pl.ANY),
            pl.BlockSpec((1, 1, D), lambda e, b: (e, 0, 0)),
        ],
        out_specs=pl.BlockSpec((T, D), lambda e, b: (0, 0)),
        out_shape=jax.ShapeDtypeStruct((T, D), jnp.float32),
        scratch_shapes=[pltpu.VMEM((D, F), jnp.float32),
                        pltpu.VMEM((F, D), jnp.float32),
                        pltpu.VMEM((SBLK, F), jnp.float32),
                        pltpu.SemaphoreType.DMA,
                        pltpu.SemaphoreType.DMA],
        compiler_params=pltpu.CompilerParams(
            vmem_limit_bytes=60 * 1024 * 1024),
    )(x, W1, b1[:, None, :], W2, b2[:, None, :])


def _routing_kernel(x_ref, gw_ref, gb_ref, p0_ref, p1_ref, w0_ref, w1_ref,
                    eidx_ref):
    T = x_ref.shape[0]
    E = gw_ref.shape[1]
    blk = BLK
    logits = jnp.dot(x_ref[...], gw_ref[...],
                     preferred_element_type=jnp.float32) + gb_ref[0]
    m = jnp.max(logits, axis=1, keepdims=True)
    p = jnp.exp(logits - m)
    p = p / jnp.sum(p, axis=1, keepdims=True)

    iota_e = lax.broadcasted_iota(jnp.int32, (T, E), 1)
    m1 = jnp.max(p, axis=1, keepdims=True)
    i1 = jnp.min(jnp.where(p == m1, iota_e, E), axis=1, keepdims=True)
    p2 = jnp.where(iota_e == i1, -1.0, p)
    m2 = jnp.max(p2, axis=1, keepdims=True)
    i2 = jnp.min(jnp.where(p2 == m2, iota_e, E), axis=1, keepdims=True)

    oh1 = (iota_e == i1)
    oh2 = (iota_e == i2)
    oh = oh1.astype(jnp.int32) + oh2.astype(jnp.int32)
    c = oh
    s = 1
    while s < T:
        c = c + jnp.concatenate(
            [jnp.zeros((s, E), jnp.int32), c[:T - s]], axis=0)
        s *= 2
    counts = c[T - 1:T]
    blocks = (counts + blk - 1) // blk
    cumblocks = blocks
    s = 1
    while s < E:
        cumblocks = cumblocks + jnp.concatenate(
            [jnp.zeros((1, s), jnp.int32), cumblocks[:, :E - s]], axis=1)
        s *= 2
    padded_off = (cumblocks - blocks) * blk
    rank1 = jnp.sum(jnp.where(oh1, c, 0), axis=1, keepdims=True) - 1
    rank2 = jnp.sum(jnp.where(oh2, c, 0), axis=1, keepdims=True) - 1
    off1 = jnp.sum(jnp.where(oh1, padded_off, 0), axis=1, keepdims=True)
    off2 = jnp.sum(jnp.where(oh2, padded_off, 0), axis=1, keepdims=True)
    p0_ref[...] = off1 + rank1
    p1_ref[...] = off2 + rank2
    w0_ref[...] = jnp.broadcast_to(m1, w0_ref.shape)
    w1_ref[...] = jnp.broadcast_to(m2, w1_ref.shape)

    nbp = eidx_ref.shape[0]
    bio = lax.broadcasted_iota(jnp.int32, (nbp, E), 0)
    cb = jnp.broadcast_to(cumblocks, (nbp, E))
    eidx = jnp.sum((cb <= bio).astype(jnp.int32), axis=1, keepdims=True)
    eidx_ref[...] = jnp.minimum(eidx, E - 1)


def _routing(x, gate_W, gate_b, nbp):
    T = x.shape[0]
    E = gate_W.shape[1]
    return pl.pallas_call(
        _routing_kernel,
        out_shape=[jax.ShapeDtypeStruct((T, 1), jnp.int32),
                   jax.ShapeDtypeStruct((T, 1), jnp.int32),
                   jax.ShapeDtypeStruct((T, LANES), jnp.float32),
                   jax.ShapeDtypeStruct((T, LANES), jnp.float32),
                   jax.ShapeDtypeStruct((nbp, 1), jnp.int32)],
    )(x, gate_W, gate_b[None, :])


def _sc_dispatch(x, pos0, pos1, np_r):
    T, D = x.shape
    tpw = T // NW
    mesh = plsc.VectorSubcoreMesh(core_axis_name="c", subcore_axis_name="s")

    @functools.partial(
        pl.kernel, mesh=mesh,
        out_type=jax.ShapeDtypeStruct((np_r, D), jnp.float32),
        scratch_types=[pltpu.VMEM((tpw,), jnp.int32),
                       pltpu.VMEM((tpw,), jnp.int32),
                       pltpu.VMEM((tpw, D), jnp.float32),
                       pltpu.SemaphoreType.DMA],
    )
    def k(x_hbm, p0_hbm, p1_hbm, xp_hbm, i0_v, i1_v, rows_v, sem):
        wid = lax.axis_index("s") * NC + lax.axis_index("c")
        base = wid * tpw
        pltpu.sync_copy(p0_hbm.at[pl.ds(base, tpw)], i0_v)
        pltpu.sync_copy(p1_hbm.at[pl.ds(base, tpw)], i1_v)
        pltpu.sync_copy(x_hbm.at[pl.ds(base, tpw)], rows_v)
        pltpu.async_copy(rows_v, xp_hbm.at[i0_v], sem).wait()
        pltpu.async_copy(rows_v, xp_hbm.at[i1_v], sem).wait()

    return k(x, pos0, pos1)


def _sc_combine(y_r, y_s, pos0, pos1, w0, w1):
    T, D = y_s.shape
    tpw = T // NW
    ch = 16
    mesh = plsc.VectorSubcoreMesh(core_axis_name="c", subcore_axis_name="s")

    @functools.partial(
        pl.kernel, mesh=mesh,
        out_type=jax.ShapeDtypeStruct((T, D), jnp.float32),
        scratch_types=[pltpu.VMEM((tpw,), jnp.int32),
                       pltpu.VMEM((tpw,), jnp.int32),
                       pltpu.VMEM((tpw, LANES), jnp.float32),
                       pltpu.VMEM((tpw, LANES), jnp.float32),
                       pltpu.VMEM((ch, D), jnp.float32),
                       pltpu.VMEM((ch, D), jnp.float32),
                       pltpu.VMEM((ch, D), jnp.float32),
                       pltpu.SemaphoreType.DMA],
    )
    def k(yr_hbm, ys_hbm, p0_hbm, p1_hbm, w0_hbm, w1_hbm, out_hbm,
          i0_v, i1_v, w0_v, w1_v, r0_v, r1_v, ys_v, sem):
        wid = lax.axis_index("s") * NC + lax.axis_index("c")
        base = wid * tpw
        pltpu.sync_copy(p0_hbm.at[pl.ds(base, tpw)], i0_v)
        pltpu.sync_copy(p1_hbm.at[pl.ds(base, tpw)], i1_v)
        pltpu.sync_copy(w0_hbm.at[pl.ds(base, tpw)], w0_v)
        pltpu.sync_copy(w1_hbm.at[pl.ds(base, tpw)], w1_v)
        for c in range(tpw // ch):
            off = c * ch
            pltpu.async_copy(yr_hbm.at[i0_v.at[pl.ds(off, ch)]], r0_v,
                             sem).wait()
            pltpu.async_copy(yr_hbm.at[i1_v.at[pl.ds(off, ch)]], r1_v,
                             sem).wait()
            pltpu.sync_copy(ys_hbm.at[pl.ds(base + off, ch)], ys_v)
            unroll = 8
            for r in range(ch):
                a0 = w0_v[off + r]
                a1 = w1_v[off + r]

                def body(j, _):
                    for u in range(unroll):
                        jj = (j * unroll + u) * LANES
                        r0_v[r, pl.ds(jj, LANES)] = (
                            a0 * r0_v[r, pl.ds(jj, LANES)]
                            + a1 * r1_v[r, pl.ds(jj, LANES)]
                            + ys_v[r, pl.ds(jj, LANES)])
                    return 0

                lax.fori_loop(0, D // (LANES * unroll), body, 0)
            pltpu.sync_copy(r0_v, out_hbm.at[pl.ds(base + off, ch)])

    return k(y_r, y_s, pos0, pos1, w0, w1)


def kernel(x, shared_W1, shared_b1, shared_W2, shared_b2, routed_W1,
           routed_b1, routed_W2, routed_b2, gate_W, gate_b):
    T, D = x.shape
    E = routed_W1.shape[0]
    nb_r = (T * TOP_K) // BLK + E - 1
    np_r = nb_r * BLK

    p0c, p1c, w0, w1, eidxc = _routing(x, gate_W, gate_b, nb_r)
    pos0 = p0c.reshape(T)
    pos1 = p1c.reshape(T)
    eidx = eidxc.reshape(nb_r)

    xp = _sc_dispatch(x, pos0, pos1, np_r)

    y_r = _routed_ffn(xp, routed_W1, routed_b1, routed_W2, routed_b2,
                      eidx, nb_r, BLK)
    y_s = _shared_ffn(x, shared_W1, shared_b1, shared_W2, shared_b2)

    return _sc_combine(y_r, y_s, pos0, pos1, w0, w1)


# speedup vs baseline: 1.8674x; 1.0299x over previous
import functools

import jax
import jax.numpy as jnp
from jax import lax
from jax.experimental import pallas as pl
from jax.experimental.pallas import tpu as pltpu
from jax.experimental.pallas import tpu_sc as plsc

TOP_K = 2
BLK = 256
SBLK = 512
NC = 2
NS = 16
NW = NC * NS
LANES = 16


def _routed_kernel(eidx_ref, xp_ref, W1_ref, b1_ref, W2_ref,
                   b2_ref, y_ref, w1c_ref, w2c_ref, h_ref, sem1, sem2):
    b = pl.program_id(0)
    e = eidx_ref[b]
    prev = eidx_ref[jnp.maximum(b - 1, 0)]
    need_load = jnp.logical_or(b == 0, e != prev)

    @pl.when(need_load)
    def _():
        pltpu.make_async_copy(W1_ref.at[e], w1c_ref, sem1).start()
        pltpu.make_async_copy(W2_ref.at[e], w2c_ref, sem2).start()

    @pl.when(need_load)
    def _():
        pltpu.make_async_copy(W1_ref.at[e], w1c_ref, sem1).wait()

    h = jnp.dot(xp_ref[...], w1c_ref[...], preferred_element_type=jnp.float32)
    h_ref[...] = jax.nn.gelu(h + b1_ref[0])

    @pl.when(need_load)
    def _():
        pltpu.make_async_copy(W2_ref.at[e], w2c_ref, sem2).wait()

    y = jnp.dot(h_ref[...], w2c_ref[...], preferred_element_type=jnp.float32)
    y_ref[...] = y + b2_ref[0]


def _shared_kernel(x_ref, W1_ref, b1_ref, W2_ref, b2_ref, o_ref,
                   w1c_ref, w2c_ref, h_ref, sem1, sem2):
    e = pl.program_id(0)
    b = pl.program_id(1)
    B = x_ref.shape[0]
    need_load = b == 0

    @pl.when(need_load)
    def _():
        pltpu.make_async_copy(W1_ref.at[e], w1c_ref, sem1).start()
        pltpu.make_async_copy(W2_ref.at[e], w2c_ref, sem2).start()

    @pl.when(need_load)
    def _():
        pltpu.make_async_copy(W1_ref.at[e], w1c_ref, sem1).wait()

    h = jnp.dot(x_ref[...], w1c_ref[...], preferred_element_type=jnp.float32)
    h_ref[...] = jax.nn.gelu(h + b1_ref[0])

    @pl.when(need_load)
    def _():
        pltpu.make_async_copy(W2_ref.at[e], w2c_ref, sem2).wait()

    y = jnp.dot(h_ref[...], w2c_ref[...], preferred_element_type=jnp.float32)
    y = y + b2_ref[0]
    row = pl.multiple_of(b * B, B)

    @pl.when(e == 0)
    def _():
        o_ref[pl.ds(row, B), :] = y

    @pl.when(e > 0)
    def _():
        o_ref[pl.ds(row, B), :] = o_ref[pl.ds(row, B), :] + y


def _routed_ffn(xp, W1, b1, W2, b2, eidx, nb, blk):
    D = xp.shape[1]
    F = W1.shape[2]
    grid_spec = pltpu.PrefetchScalarGridSpec(
        num_scalar_prefetch=1,
        grid=(nb,),
        in_specs=[
            pl.BlockSpec((blk, D), lambda b, e: (b, 0)),
            pl.BlockSpec(memory_space=pl.ANY),
            pl.BlockSpec((1, 1, F), lambda b, e: (e[b], 0, 0)),
            pl.BlockSpec(memory_space=pl.ANY),
            pl.BlockSpec((1, 1, D), lambda b, e: (e[b], 0, 0)),
        ],
        out_specs=pl.BlockSpec((blk, D), lambda b, e: (b, 0)),
        scratch_shapes=[pltpu.VMEM((D, F), jnp.float32),
                        pltpu.VMEM((F, D), jnp.float32),
                        pltpu.VMEM((blk, F), jnp.float32),
                        pltpu.SemaphoreType.DMA,
                        pltpu.SemaphoreType.DMA],
    )
    return pl.pallas_call(
        _routed_kernel,
        grid_spec=grid_spec,
        out_shape=jax.ShapeDtypeStruct((nb * blk, D), jnp.float32),
        compiler_params=pltpu.CompilerParams(
            vmem_limit_bytes=60 * 1024 * 1024),
    )(eidx, xp, W1, b1[:, None, :], W2, b2[:, None, :])


def _shared_ffn(x, W1, b1, W2, b2):
    T, D = x.shape
    S, _, F = W1.shape
    nb = T // SBLK
    return pl.pallas_call(
        _shared_kernel,
        grid=(S, nb),
        in_specs=[
            pl.BlockSpec((SBLK, D), lambda e, b: (b, 0)),
            pl.BlockSpec(memory_space=pl.ANY),
            pl.BlockSpec((1, 1, F), lambda e, b: (e, 0, 0)),
            pl.BlockSpec(memory_space=pl.ANY),
            pl.BlockSpec((1, 1, D), lambda e, b: (e, 0, 0)),
        ],
        out_specs=pl.BlockSpec((T, D), lambda e, b: (0, 0)),
        out_shape=jax.ShapeDtypeStruct((T, D), jnp.float32),
        scratch_shapes=[pltpu.VMEM((D, F), jnp.float32),
                        pltpu.VMEM((F, D), jnp.float32),
                        pltpu.VMEM((SBLK, F), jnp.float32),
                        pltpu.SemaphoreType.DMA,
                        pltpu.SemaphoreType.DMA],
        compiler_params=pltpu.CompilerParams(
            vmem_limit_bytes=60 * 1024 * 1024),
    )(x, W1, b1[:, None, :], W2, b2[:, None, :])


def _routing_kernel(x_ref, gw_ref, gb_ref, p0_ref, p1_ref, w0_ref, w1_ref,
                    eidx_ref):
    T = x_ref.shape[0]
    E = gw_ref.shape[1]
    blk = BLK
    logits = jnp.dot(x_ref[...], gw_ref[...],
                     preferred_element_type=jnp.float32) + gb_ref[0]
    m = jnp.max(logits, axis=1, keepdims=True)
    p = jnp.exp(logits - m)
    p = p / jnp.sum(p, axis=1, keepdims=True)

    iota_e = lax.broadcasted_iota(jnp.int32, (T, E), 1)
    m1 = jnp.max(p, axis=1, keepdims=True)
    i1 = jnp.min(jnp.where(p == m1, iota_e, E), axis=1, keepdims=True)
    p2 = jnp.where(iota_e == i1, -1.0, p)
    m2 = jnp.max(p2, axis=1, keepdims=True)
    i2 = jnp.min(jnp.where(p2 == m2, iota_e, E), axis=1, keepdims=True)

    oh1 = (iota_e == i1)
    oh2 = (iota_e == i2)
    oh = oh1.astype(jnp.int32) + oh2.astype(jnp.int32)
    c = oh
    s = 1
    while s < T:
        c = c + jnp.concatenate(
            [jnp.zeros((s, E), jnp.int32), c[:T - s]], axis=0)
        s *= 2
    counts = c[T - 1:T]
    blocks = (counts + blk - 1) // blk
    cumblocks = blocks
    s = 1
    while s < E:
        cumblocks = cumblocks + jnp.concatenate(
            [jnp.zeros((1, s), jnp.int32), cumblocks[:, :E - s]], axis=1)
        s *= 2
    padded_off = (cumblocks - blocks) * blk
    rank1 = jnp.sum(jnp.where(oh1, c, 0), axis=1, keepdims=True) - 1
    rank2 = jnp.sum(jnp.where(oh2, c, 0), axis=1, keepdims=True) - 1
    off1 = jnp.sum(jnp.where(oh1, padded_off, 0), axis=1, keepdims=True)
    off2 = jnp.sum(jnp.where(oh2, padded_off, 0), axis=1, keepdims=True)
    p0_ref[...] = off1 + rank1
    p1_ref[...] = off2 + rank2
    w0_ref[...] = jnp.broadcast_to(m1, w0_ref.shape)
    w1_ref[...] = jnp.broadcast_to(m2, w1_ref.shape)

    nbp = eidx_ref.shape[0]
    bio = lax.broadcasted_iota(jnp.int32, (nbp, E), 0)
    cb = jnp.broadcast_to(cumblocks, (nbp, E))
    eidx = jnp.sum((cb <= bio).astype(jnp.int32), axis=1, keepdims=True)
    eidx_ref[...] = jnp.minimum(eidx, E - 1)


def _routing(x, gate_W, gate_b, nbp):
    T = x.shape[0]
    E = gate_W.shape[1]
    return pl.pallas_call(
        _routing_kernel,
        out_shape=[jax.ShapeDtypeStruct((T, 1), jnp.int32),
                   jax.ShapeDtypeStruct((T, 1), jnp.int32),
                   jax.ShapeDtypeStruct((T, LANES), jnp.float32),
                   jax.ShapeDtypeStruct((T, LANES), jnp.float32),
                   jax.ShapeDtypeStruct((nbp, 1), jnp.int32)],
    )(x, gate_W, gate_b[None, :])


def _sc_dispatch(x, pos0, pos1, np_r):
    T, D = x.shape
    tpw = T // NW
    mesh = plsc.VectorSubcoreMesh(core_axis_name="c", subcore_axis_name="s")

    @functools.partial(
        pl.kernel, mesh=mesh,
        out_type=jax.ShapeDtypeStruct((np_r, D), jnp.float32),
        scratch_types=[pltpu.VMEM((tpw,), jnp.int32),
                       pltpu.VMEM((tpw,), jnp.int32),
                       pltpu.VMEM((tpw, D), jnp.float32),
                       pltpu.SemaphoreType.DMA],
    )
    def k(x_hbm, p0_hbm, p1_hbm, xp_hbm, i0_v, i1_v, rows_v, sem):
        wid = lax.axis_index("s") * NC + lax.axis_index("c")
        base = wid * tpw
        pltpu.sync_copy(p0_hbm.at[pl.ds(base, tpw)], i0_v)
        pltpu.sync_copy(p1_hbm.at[pl.ds(base, tpw)], i1_v)
        pltpu.sync_copy(x_hbm.at[pl.ds(base, tpw)], rows_v)
        pltpu.async_copy(rows_v, xp_hbm.at[i0_v], sem).wait()
        pltpu.async_copy(rows_v, xp_hbm.at[i1_v], sem).wait()

    return k(x, pos0, pos1)


def _sc_combine(y_r, y_s, pos0, pos1, w0, w1):
    T, D = y_s.shape
    tpw = T // NW
    ch = 32
    mesh = plsc.VectorSubcoreMesh(core_axis_name="c", subcore_axis_name="s")

    @functools.partial(
        pl.kernel, mesh=mesh,
        out_type=jax.ShapeDtypeStruct((T, D), jnp.float32),
        scratch_types=[pltpu.VMEM((tpw,), jnp.int32),
                       pltpu.VMEM((tpw,), jnp.int32),
                       pltpu.VMEM((tpw, LANES), jnp.float32),
                       pltpu.VMEM((tpw, LANES), jnp.float32),
                       pltpu.VMEM((ch, D), jnp.float32),
                       pltpu.VMEM((ch, D), jnp.float32),
                       pltpu.VMEM((ch, D), jnp.float32),
                       pltpu.SemaphoreType.DMA],
    )
    def k(yr_hbm, ys_hbm, p0_hbm, p1_hbm, w0_hbm, w1_hbm, out_hbm,
          i0_v, i1_v, w0_v, w1_v, r0_v, r1_v, ys_v, sem):
        wid = lax.axis_index("s") * NC + lax.axis_index("c")
        base = wid * tpw
        pltpu.sync_copy(p0_hbm.at[pl.ds(base, tpw)], i0_v)
        pltpu.sync_copy(p1_hbm.at[pl.ds(base, tpw)], i1_v)
        pltpu.sync_copy(w0_hbm.at[pl.ds(base, tpw)], w0_v)
        pltpu.sync_copy(w1_hbm.at[pl.ds(base, tpw)], w1_v)
        for c in range(tpw // ch):
            off = c * ch
            pltpu.async_copy(yr_hbm.at[i0_v.at[pl.ds(off, ch)]], r0_v,
                             sem).wait()
            pltpu.async_copy(yr_hbm.at[i1_v.at[pl.ds(off, ch)]], r1_v,
                             sem).wait()
            pltpu.sync_copy(ys_hbm.at[pl.ds(base + off, ch)], ys_v)
            unroll = 8
            for r in range(ch):
                a0 = w0_v[off + r]
                a1 = w1_v[off + r]

                def body(j, _):
                    for u in range(unroll):
                        jj = (j * unroll + u) * LANES
                        r0_v[r, pl.ds(jj, LANES)] = (
                            a0 * r0_v[r, pl.ds(jj, LANES)]
                            + a1 * r1_v[r, pl.ds(jj, LANES)]
                            + ys_v[r, pl.ds(jj, LANES)])
                    return 0

                lax.fori_loop(0, D // (LANES * unroll), body, 0)
            pltpu.sync_copy(r0_v, out_hbm.at[pl.ds(base + off, ch)])

    return k(y_r, y_s, pos0, pos1, w0, w1)


def kernel(x, shared_W1, shared_b1, shared_W2, shared_b2, routed_W1,
           routed_b1, routed_W2, routed_b2, gate_W, gate_b):
    T, D = x.shape
    E = routed_W1.shape[0]
    nb_r = (T * TOP_K) // BLK + E - 1
    np_r = nb_r * BLK

    p0c, p1c, w0, w1, eidxc = _routing(x, gate_W, gate_b, nb_r)
    pos0 = p0c.reshape(T)
    pos1 = p1c.reshape(T)
    eidx = eidxc.reshape(nb_r)

    xp = _sc_dispatch(x, pos0, pos1, np_r)

    y_r = _routed_ffn(xp, routed_W1, routed_b1, routed_W2, routed_b2,
                      eidx, nb_r, BLK)
    y_s = _shared_ffn(x, shared_W1, shared_b1, shared_W2, shared_b2)

    return _sc_combine(y_r, y_s, pos0, pos1, w0, w1)
